# baseline (speedup 1.0000x reference)
import functools

import jax
import jax.numpy as jnp
from jax import lax
from jax.experimental import pallas as pl
from jax.experimental.pallas import tpu as pltpu
from jax.experimental.pallas import tpu_sc as plsc

N = 10000
D = 128
E = 160000
NET = 8
TD = 3 * D
LAYER_TIMESTEPS = (8, 2, 8, 2)
RESIDUALS = {1: 0, 3: 2}

NC = 2
NS = 16
NW = NC * NS
ETC = NET // NC
CH = 128
RNG = 632
RLAST = N - (NW // 2 - 1) * RNG
K2 = 13
CAP = K2 * CH
ACC = RNG + 8

GC = 3
GPAD = NW * GC * CH

BN = 1000


def _mesh():
    return plsc.VectorSubcoreMesh(core_axis_name="c", subcore_axis_name="s",
                                  num_cores=NC, num_subcores=NS)


def _ordered_scatter(hwb_flat, srcs, dstl):
    @functools.partial(
        pl.kernel,
        out_type=jax.ShapeDtypeStruct((NET, N, D), jnp.float32),
        mesh=_mesh(),
        scratch_types=[
            pltpu.VMEM((ACC, D), jnp.float32),
            pltpu.VMEM((K2, CH), jnp.int32),
            pltpu.VMEM((K2, CH), jnp.int32),
            pltpu.VMEM((CH, D), jnp.float32),
            pltpu.SemaphoreType.DMA,
        ],
    )
    def kern(hwb_hbm, srcs_hbm, dstl_hbm, out_hbm,
             acc, src_v, dst_v, rows_v, sem):
        c = lax.axis_index("c")
        s = lax.axis_index("s")
        base = s * RNG
        for etl in range(ETC):
            et = c * ETC + etl

            def zero(j, carry):
                for l in range(D // 16):
                    acc[j, pl.ds(l * 16, 16)] = jnp.zeros((16,), jnp.float32)
                return carry
            lax.fori_loop(0, ACC, zero, 0)

            pltpu.sync_copy(srcs_hbm.at[et, s], src_v)
            pltpu.sync_copy(dstl_hbm.at[et, s], dst_v)

            def chunk(k, carry):
                pltpu.async_copy(hwb_hbm.at[src_v.at[k]], rows_v, sem).wait()

                def add16(i, carry2):
                    dvec = dst_v[k, pl.ds(i * 16, 16)]
                    for lane in range(16):
                        r = dvec[lane]
                        e = i * 16 + lane
                        for l in range(D // 16):
                            sl = pl.ds(l * 16, 16)
                            acc[r, sl] = acc[r, sl] + rows_v[e, sl]
                    return carry2
                lax.fori_loop(0, CH // 16, add16, 0)
                return carry
            lax.fori_loop(0, K2, chunk, 0)

            @pl.when(s < NS - 1)
            def _():
                pltpu.sync_copy(acc.at[pl.ds(0, RNG)],
                                out_hbm.at[et, pl.ds(base, RNG)])
            @pl.when(s == NS - 1)
            def _():
                pltpu.sync_copy(acc.at[pl.ds(0, RLAST)],
                                out_hbm.at[et, pl.ds(base, RLAST)])

    return kern(hwb_flat, srcs, dstl)


def _embed_gather(table, idx):
    @functools.partial(
        pl.kernel,
        out_type=jax.ShapeDtypeStruct((GPAD, D), jnp.float32),
        mesh=_mesh(),
        scratch_types=[
            pltpu.VMEM((GC, CH), jnp.int32),
            pltpu.VMEM((CH, D), jnp.float32),
            pltpu.SemaphoreType.DMA,
        ],
    )
    def kern(table_hbm, idx_hbm, out_hbm, idx_v, rows_v, sem):
        c = lax.axis_index("c")
        s = lax.axis_index("s")
        base = (c * NS + s) * GC * CH
        pltpu.sync_copy(idx_hbm.at[c, s], idx_v)
        for k in range(GC):
            pltpu.async_copy(table_hbm.at[idx_v.at[k]], rows_v, sem).wait()
            pltpu.sync_copy(rows_v, out_hbm.at[pl.ds(base + k * CH, CH)])

    return kern(table, idx)


def _messages(h, W, B):
    def body(h_ref, w_ref, b_ref, o_ref):
        for et in range(NET):
            o_ref[et] = jnp.dot(h_ref[...], w_ref[et],
                                preferred_element_type=jnp.float32) + b_ref[et]

    return pl.pallas_call(
        body,
        grid=(N // BN,),
        in_specs=[pl.BlockSpec((BN, D), lambda i: (i, 0)),
                  pl.BlockSpec((NET, D, D), lambda i: (0, 0, 0)),
                  pl.BlockSpec((NET, 1, D), lambda i: (0, 0, 0))],
        out_specs=pl.BlockSpec((NET, BN, D), lambda i: (0, i, 0)),
        out_shape=jax.ShapeDtypeStruct((NET, N, D), jnp.float32),
    )(h, W, B)


def _gru_step(S, h, res, Wi, Wh, bi2, bh2):
    has_res = res is not None

    def body(*refs):
        if has_res:
            (s_ref, h_ref, r_ref, wi_ref, wh_ref, bi_ref, bh_ref, out_ref) = refs
        else:
            (s_ref, h_ref, wi_ref, wh_ref, bi_ref, bh_ref, out_ref) = refs
        inc = s_ref[0]
        for et in range(1, NET):
            inc = inc + s_ref[et]
        h_blk = h_ref[...]
        x = jnp.concatenate([r_ref[...], inc], axis=-1) if has_res else inc
        gi = jnp.dot(x, wi_ref[...],
                     preferred_element_type=jnp.float32) + bi_ref[...]
        gh = jnp.dot(h_blk, wh_ref[...],
                     preferred_element_type=jnp.float32) + bh_ref[...]
        r = jax.nn.sigmoid(gi[:, :D] + gh[:, :D])
        z = jax.nn.sigmoid(gi[:, D:2 * D] + gh[:, D:2 * D])
        n = jnp.tanh(gi[:, 2 * D:] + r * gh[:, 2 * D:])
        out_ref[...] = (1.0 - z) * n + z * h_blk

    in_dim = 2 * D if has_res else D
    in_specs = [
        pl.BlockSpec((NET, BN, D), lambda i: (0, i, 0)),
        pl.BlockSpec((BN, D), lambda i: (i, 0)),
    ]
    args = [S, h]
    if has_res:
        in_specs.append(pl.BlockSpec((BN, D), lambda i: (i, 0)))
        args.append(res)
    in_specs += [
        pl.BlockSpec((in_dim, TD), lambda i: (0, 0)),
        pl.BlockSpec((D, TD), lambda i: (0, 0)),
        pl.BlockSpec((1, TD), lambda i: (0, 0)),
        pl.BlockSpec((1, TD), lambda i: (0, 0)),
    ]
    args += [Wi, Wh, bi2, bh2]
    return pl.pallas_call(
        body,
        grid=(N // BN,),
        in_specs=in_specs,
        out_specs=pl.BlockSpec((BN, D), lambda i: (i, 0)),
        out_shape=jax.ShapeDtypeStruct((N, D), jnp.float32),
    )(*args)


def kernel(node_type_ids, edge_index, embedding, msg_W, msg_b,
           gru_Wi_0, gru_Wh_0, gru_bi_0, gru_bh_0,
           gru_Wi_1, gru_Wh_1, gru_bi_1, gru_bh_1,
           gru_Wi_2, gru_Wh_2, gru_bi_2, gru_bh_2,
           gru_Wi_3, gru_Wh_3, gru_bi_3, gru_bh_3):
    gru_params = [
        (gru_Wi_0, gru_Wh_0, gru_bi_0, gru_bh_0),
        (gru_Wi_1, gru_Wh_1, gru_bi_1, gru_bh_1),
        (gru_Wi_2, gru_Wh_2, gru_bi_2, gru_bh_2),
        (gru_Wi_3, gru_Wh_3, gru_bi_3, gru_bh_3),
    ]
    src = edge_index[0].astype(jnp.int32).reshape(NET, E // NET)
    dst = edge_index[1].astype(jnp.int32).reshape(NET, E // NET)

    tile = jnp.minimum(dst // RNG, NS - 1)
    order = jnp.argsort(tile, axis=1, stable=True)
    src_s = jnp.take_along_axis(src, order, axis=1)
    dst_s = jnp.take_along_axis(dst, order, axis=1)
    tile_s = jnp.take_along_axis(tile, order, axis=1)
    e_per = E // NET
    onehot = (tile_s[:, :, None] == jnp.arange(NS)[None, None, :])
    cnts = jnp.sum(onehot, axis=1).astype(jnp.int32)
    starts = jnp.concatenate(
        [jnp.zeros((NET, 1), jnp.int32), jnp.cumsum(cnts, axis=1)[:, :-1]], axis=1)
    pos = starts[:, :, None] + jnp.arange(CAP, dtype=jnp.int32)[None, None, :]
    valid = pos < (starts + cnts)[:, :, None]
    posc = jnp.clip(pos, 0, e_per - 1)
    src_b = jnp.where(valid, jnp.take_along_axis(
        src_s[:, None, :].repeat(NS, 1).reshape(NET, NS, e_per), posc, axis=2), 0)
    dst_b = jnp.where(valid, jnp.take_along_axis(
        dst_s[:, None, :].repeat(NS, 1).reshape(NET, NS, e_per), posc, axis=2), -1)
    srcs = (src_b + (jnp.arange(NET, dtype=jnp.int32) * N)[:, None, None]
            ).reshape(NET, NS, K2, CH)
    base = (jnp.arange(NS, dtype=jnp.int32) * RNG)[None, :, None]
    dstl = jnp.where(dst_b >= 0, dst_b - base, RNG).astype(jnp.int32
                     ).reshape(NET, NS, K2, CH)

    idx = jnp.concatenate(
        [node_type_ids.astype(jnp.int32), jnp.zeros((GPAD - N,), jnp.int32)]
    ).reshape(NC, NS, GC, CH)
    h = _embed_gather(embedding, idx)[:N]
    srcs, h = lax.optimization_barrier((srcs, h))

    states = [h]
    for layer in range(4):
        Wi, Wh, bi, bh = gru_params[layer]
        bi2 = bi.reshape(1, TD)
        bh2 = bh.reshape(1, TD)
        res_idx = RESIDUALS.get(layer)
        res = states[res_idx] if res_idx is not None else None
        W_l = msg_W[layer]
        B_l = msg_b[layer].reshape(NET, 1, D)
        for _ in range(LAYER_TIMESTEPS[layer]):
            Hwb = _messages(h, W_l, B_l).reshape(NET * N, D)
            S = _ordered_scatter(Hwb, srcs, dstl)
            h = _gru_step(S, h, res, Wi, Wh, bi2, bh2)
        states.append(h)
    return h
